# baseline (speedup 1.0000x reference)
import functools

import jax
import jax.numpy as jnp
from jax import lax
from jax.experimental import pallas as pl
from jax.experimental.pallas import tpu as pltpu
from jax.experimental.pallas import tpu_sc as plsc

N_E = 1024
E_DIM = 64
BETA = 0.25
B_TOTAL = 8 * 1024
BLOCK_B = 1024
N_BLOCKS = B_TOTAL // BLOCK_B

_NC, _NS = 2, 16
_NW = _NC * _NS
_BPW = B_TOTAL // _NW


def _xla_style_rowsq(x):
    t = x * x
    p = t[:, 0:8]
    for i in range(1, 8):
        p = p + t[:, 8 * i:8 * i + 8]
    q04 = p[:, 0:1] + p[:, 4:5]
    q26 = p[:, 2:3] + p[:, 6:7]
    q15 = p[:, 1:2] + p[:, 5:6]
    q37 = p[:, 3:4] + p[:, 7:8]
    return (q04 + q26) + (q15 + q37)


def _vq_body(z_ref, emb_ref, idx_ref, loss_ref, perp_ref, counts_ref):
    i = pl.program_id(0)
    zb = z_ref[...]
    emb = emb_ref[...]

    z2 = _xla_style_rowsq(zb)
    e2 = _xla_style_rowsq(emb).reshape(1, N_E)
    s = jax.lax.dot_general(
        zb, emb, dimension_numbers=(((1,), (1,)), ((), ())))
    d = (z2 + e2) - 2.0 * s

    dmin = jnp.min(d, axis=1)
    iota = jax.lax.broadcasted_iota(jnp.int32, (BLOCK_B, N_E), 1)
    idx = jnp.min(jnp.where(d == dmin[:, None], iota, N_E), axis=1)
    idx_ref[0, 0, :] = idx

    oh = (iota == idx[:, None]).astype(jnp.float32)

    @pl.when(i == 0)
    def _init():
        loss_ref[...] = jnp.zeros_like(loss_ref)
        perp_ref[...] = jnp.zeros_like(perp_ref)
        counts_ref[...] = jnp.zeros_like(counts_ref)

    loss_ref[...] += jnp.sum(dmin.reshape(-1, 128), axis=0)[None, :]
    counts_ref[...] += jnp.sum(oh, axis=0).reshape(8, 128)

    @pl.when(i == pl.num_programs(0) - 1)
    def _finalize():
        lane0 = jax.lax.broadcasted_iota(jnp.int32, (1, 128), 1) == 0
        total = jnp.sum(loss_ref[...])
        loss_val = (1.0 + BETA) * total / float(B_TOTAL * E_DIM)
        loss_ref[...] = jnp.where(lane0, loss_val, 0.0)
        p = counts_ref[...] / float(B_TOTAL)
        ent = jnp.sum(p * jnp.log(p + 1e-10))
        perp_ref[...] = jnp.where(lane0, jnp.exp(-ent), 0.0)


@jax.jit
def _vq_call(z_flat, embeddings):
    out_shapes = (
        jax.ShapeDtypeStruct((N_BLOCKS, 1, BLOCK_B), jnp.int32),
        jax.ShapeDtypeStruct((1, 128), jnp.float32),
        jax.ShapeDtypeStruct((1, 128), jnp.float32),
        jax.ShapeDtypeStruct((8, 128), jnp.float32),
    )
    return pl.pallas_call(
        _vq_body,
        grid=(N_BLOCKS,),
        in_specs=[
            pl.BlockSpec((BLOCK_B, E_DIM), lambda i: (i, 0)),
            pl.BlockSpec((N_E, E_DIM), lambda i: (0, 0)),
        ],
        out_specs=(
            pl.BlockSpec((1, 1, BLOCK_B), lambda i: (i, 0, 0)),
            pl.BlockSpec((1, 128), lambda i: (0, 0)),
            pl.BlockSpec((1, 128), lambda i: (0, 0)),
            pl.BlockSpec((8, 128), lambda i: (0, 0)),
        ),
        out_shape=out_shapes,
    )(z_flat, embeddings)


_CHUNK = 128
_NCHUNK = _BPW // _CHUNK


def _gather_body(emb_hbm, idx2_hbm, out_hbm, idx_v, rows_v, sem):
    wid = lax.axis_index("s") * _NC + lax.axis_index("c")
    pltpu.sync_copy(idx2_hbm.at[pl.ds(wid * _NCHUNK, _NCHUNK)], idx_v)
    copies = []
    for j in range(_NCHUNK):
        copies.append(
            pltpu.async_copy(emb_hbm.at[idx_v.at[j]], rows_v.at[j], sem))
    for c in copies:
        c.wait()
    pltpu.sync_copy(rows_v, out_hbm.at[pl.ds(wid * _NCHUNK, _NCHUNK)])


@jax.jit
def _sc_gather(embeddings, idx2):
    mesh = plsc.VectorSubcoreMesh(
        core_axis_name="c", subcore_axis_name="s",
        num_cores=_NC, num_subcores=_NS)
    k = functools.partial(
        pl.kernel,
        out_type=jax.ShapeDtypeStruct((B_TOTAL // _CHUNK, _CHUNK, E_DIM),
                                      jnp.float32),
        mesh=mesh,
        compiler_params=pltpu.CompilerParams(use_tc_tiling_on_sc=False),
        scratch_types=[
            pltpu.VMEM((_NCHUNK, _CHUNK), jnp.int32),
            pltpu.VMEM((_NCHUNK, _CHUNK, E_DIM), jnp.float32),
            pltpu.SemaphoreType.DMA,
        ],
    )(_gather_body)
    return k(embeddings, idx2)


def kernel(z, embeddings):
    z_flat = z.reshape(-1, z.shape[-1])
    idx3, loss_v, perp_v, _counts = _vq_call(z_flat, embeddings)
    idx2 = idx3.reshape(-1, _CHUNK)
    zq3 = _sc_gather(embeddings, idx2)
    indices = idx3.reshape(z.shape[:-1])
    z_q = zq3.reshape(z.shape)
    return (z_q, indices, loss_v[0, 0], perp_v[0, 0])


# speedup vs baseline: 1.0261x; 1.0261x over previous
import jax
import jax.numpy as jnp
from jax.experimental import pallas as pl
from jax.experimental.pallas import tpu as pltpu

N_E = 1024
E_DIM = 64
BETA = 0.25
B_TOTAL = 8 * 1024
BLOCK_B = 1024
N_BLOCKS = B_TOTAL // BLOCK_B


def _xla_style_rowsq(x):
    t = x * x
    p = t[:, 0:8]
    for i in range(1, 8):
        p = p + t[:, 8 * i:8 * i + 8]
    q04 = p[:, 0:1] + p[:, 4:5]
    q26 = p[:, 2:3] + p[:, 6:7]
    q15 = p[:, 1:2] + p[:, 5:6]
    q37 = p[:, 3:4] + p[:, 7:8]
    return (q04 + q26) + (q15 + q37)


def _vq_body(z_ref, emb_ref, zq_ref, idx_ref, loss_ref, perp_ref, counts_ref,
             e2_ref):
    i = pl.program_id(0)
    zb = z_ref[...]
    emb = emb_ref[...]

    @pl.when(i == 0)
    def _pre():
        e2_ref[...] = _xla_style_rowsq(emb).reshape(1, N_E)

    z2 = _xla_style_rowsq(zb)
    e2 = e2_ref[...]
    s = jax.lax.dot_general(
        zb, emb, dimension_numbers=(((1,), (1,)), ((), ())))
    d = (z2 + e2) - 2.0 * s

    dmin = jnp.min(d, axis=1)
    iota = jax.lax.broadcasted_iota(jnp.int32, (BLOCK_B, N_E), 1)
    idx = jnp.min(jnp.where(d == dmin[:, None], iota, N_E), axis=1)
    idx_ref[i, :] = idx

    oh = (iota == idx[:, None]).astype(jnp.float32)
    zq_ref[...] = jax.lax.dot_general(
        oh, emb, dimension_numbers=(((1,), (0,)), ((), ())))

    @pl.when(i == 0)
    def _init():
        loss_ref[...] = jnp.zeros_like(loss_ref)
        perp_ref[...] = jnp.zeros_like(perp_ref)
        counts_ref[...] = jnp.zeros_like(counts_ref)

    loss_ref[...] += jnp.sum(dmin.reshape(-1, 128), axis=0)[None, :]
    counts_ref[...] += jnp.sum(oh, axis=0).reshape(8, 128)

    @pl.when(i == pl.num_programs(0) - 1)
    def _finalize():
        lane0 = jax.lax.broadcasted_iota(jnp.int32, (1, 128), 1) == 0
        total = jnp.sum(loss_ref[...])
        loss_val = (1.0 + BETA) * total / float(B_TOTAL * E_DIM)
        loss_ref[...] = jnp.where(lane0, loss_val, 0.0)
        p = counts_ref[...] / float(B_TOTAL)
        ent = jnp.sum(p * jnp.log(p + 1e-10))
        perp_ref[...] = jnp.where(lane0, jnp.exp(-ent), 0.0)


@jax.jit
def _vq_call(z_flat, embeddings):
    out_shapes = (
        jax.ShapeDtypeStruct((B_TOTAL, E_DIM), jnp.float32),
        jax.ShapeDtypeStruct((N_BLOCKS, BLOCK_B), jnp.int32),
        jax.ShapeDtypeStruct((1, 128), jnp.float32),
        jax.ShapeDtypeStruct((1, 128), jnp.float32),
        jax.ShapeDtypeStruct((8, 128), jnp.float32),
    )
    return pl.pallas_call(
        _vq_body,
        grid=(N_BLOCKS,),
        in_specs=[
            pl.BlockSpec((BLOCK_B, E_DIM), lambda i: (i, 0)),
            pl.BlockSpec((N_E, E_DIM), lambda i: (0, 0)),
        ],
        out_specs=(
            pl.BlockSpec((BLOCK_B, E_DIM), lambda i: (i, 0)),
            pl.BlockSpec((N_BLOCKS, BLOCK_B), lambda i: (0, 0)),
            pl.BlockSpec((1, 128), lambda i: (0, 0)),
            pl.BlockSpec((1, 128), lambda i: (0, 0)),
            pl.BlockSpec((8, 128), lambda i: (0, 0)),
        ),
        out_shape=out_shapes,
        scratch_shapes=[pltpu.VMEM((1, N_E), jnp.float32)],
    )(z_flat, embeddings)


def kernel(z, embeddings):
    z_flat = z.reshape(-1, z.shape[-1])
    zq, indices, loss_v, perp_v, _counts = _vq_call(z_flat, embeddings)
    return (zq.reshape(z.shape), indices, loss_v[0, 0], perp_v[0, 0])


# speedup vs baseline: 1.3283x; 1.2945x over previous
import functools

import jax
import jax.numpy as jnp
from jax import lax
from jax.experimental import pallas as pl
from jax.experimental.pallas import tpu as pltpu
from jax.experimental.pallas import tpu_sc as plsc

N_E = 1024
E_DIM = 64
BETA = 0.25
B_TOTAL = 8 * 1024
BLOCK_B = 1024
N_BLOCKS = B_TOTAL // BLOCK_B

_NC, _NS = 2, 16
_NW = _NC * _NS
_BPW = B_TOTAL // _NW
_CHUNK = 128
_NCHUNK = _BPW // _CHUNK


def _xla_style_colsq(xt):
    t = xt * xt
    p = t[0:8]
    for i in range(1, 8):
        p = p + t[8 * i:8 * i + 8]
    q = p + pltpu.roll(p, 4, axis=0)
    r = q + pltpu.roll(q, 6, axis=0)
    f = r + pltpu.roll(r, 7, axis=0)
    return f[0:1]


def _vq_body(z_ref, emb_ref, idx_ref, idxsc_ref, loss_ref, perp_ref,
             counts_ref, e2_ref):
    i = pl.program_id(0)
    zb = z_ref[...]
    emb = emb_ref[...]

    @pl.when(i == 0)
    def _pre():
        e2_ref[...] = _xla_style_colsq(emb.T)

    z2 = _xla_style_colsq(zb.T).reshape(BLOCK_B, 1)
    s = jax.lax.dot_general(
        zb, emb, dimension_numbers=(((1,), (1,)), ((), ())))
    d = (z2 + e2_ref[...]) - 2.0 * s

    dmin = jnp.min(d, axis=1)
    iota = jax.lax.broadcasted_iota(jnp.int32, (BLOCK_B, N_E), 1)
    idx = jnp.min(jnp.where(d == dmin[:, None], iota, N_E), axis=1)
    idx_ref[i, :] = idx
    idxsc_ref[...] = idx.reshape(8, 128)

    oh = (iota == idx[:, None]).astype(jnp.float32)

    @pl.when(i == 0)
    def _init():
        loss_ref[...] = jnp.zeros_like(loss_ref)
        perp_ref[...] = jnp.zeros_like(perp_ref)
        counts_ref[...] = jnp.zeros_like(counts_ref)

    loss_ref[...] += jnp.sum(dmin.reshape(-1, 128), axis=0)[None, :]
    counts_ref[...] += jnp.sum(oh, axis=0).reshape(8, 128)

    @pl.when(i == pl.num_programs(0) - 1)
    def _finalize():
        lane0 = jax.lax.broadcasted_iota(jnp.int32, (1, 128), 1) == 0
        total = jnp.sum(loss_ref[...])
        loss_val = (1.0 + BETA) * total / float(B_TOTAL * E_DIM)
        loss_ref[...] = jnp.where(lane0, loss_val, 0.0)
        p = counts_ref[...] / float(B_TOTAL)
        ent = jnp.sum(p * jnp.log(p + 1e-10))
        perp_ref[...] = jnp.where(lane0, jnp.exp(-ent), 0.0)


@jax.jit
def _vq_call(z_flat, embeddings):
    out_shapes = (
        jax.ShapeDtypeStruct((N_BLOCKS, BLOCK_B), jnp.int32),
        jax.ShapeDtypeStruct((B_TOTAL // 128, 128), jnp.int32),
        jax.ShapeDtypeStruct((1, 128), jnp.float32),
        jax.ShapeDtypeStruct((1, 128), jnp.float32),
        jax.ShapeDtypeStruct((8, 128), jnp.float32),
    )
    return pl.pallas_call(
        _vq_body,
        grid=(N_BLOCKS,),
        in_specs=[
            pl.BlockSpec((BLOCK_B, E_DIM), lambda i: (i, 0)),
            pl.BlockSpec((N_E, E_DIM), lambda i: (0, 0)),
        ],
        out_specs=(
            pl.BlockSpec((N_BLOCKS, BLOCK_B), lambda i: (0, 0)),
            pl.BlockSpec((8, 128), lambda i: (i, 0)),
            pl.BlockSpec((1, 128), lambda i: (0, 0)),
            pl.BlockSpec((1, 128), lambda i: (0, 0)),
            pl.BlockSpec((8, 128), lambda i: (0, 0)),
        ),
        out_shape=out_shapes,
        scratch_shapes=[pltpu.VMEM((1, N_E), jnp.float32)],
    )(z_flat, embeddings)


def _gather_body(emb_hbm, idx2_hbm, out_hbm, idx_v, rows_v, sem):
    wid = lax.axis_index("s") * _NC + lax.axis_index("c")
    pltpu.sync_copy(idx2_hbm.at[pl.ds(wid * _NCHUNK, _NCHUNK)], idx_v)
    copies = []
    for j in range(_NCHUNK):
        copies.append(
            pltpu.async_copy(emb_hbm.at[idx_v.at[j]], rows_v.at[j], sem))
    for c in copies:
        c.wait()
    pltpu.sync_copy(rows_v, out_hbm.at[pl.ds(wid * _NCHUNK, _NCHUNK)])


@jax.jit
def _sc_gather(embeddings, idx2):
    mesh = plsc.VectorSubcoreMesh(
        core_axis_name="c", subcore_axis_name="s",
        num_cores=_NC, num_subcores=_NS)
    k = functools.partial(
        pl.kernel,
        out_type=jax.ShapeDtypeStruct((B_TOTAL // _CHUNK, _CHUNK, E_DIM),
                                      jnp.float32),
        mesh=mesh,
        compiler_params=pltpu.CompilerParams(use_tc_tiling_on_sc=False),
        scratch_types=[
            pltpu.VMEM((_NCHUNK, _CHUNK), jnp.int32),
            pltpu.VMEM((_NCHUNK, _CHUNK, E_DIM), jnp.float32),
            pltpu.SemaphoreType.DMA,
        ],
    )(_gather_body)
    return k(embeddings, idx2)


def kernel(z, embeddings):
    z_flat = z.reshape(-1, z.shape[-1])
    indices, idx2, loss_v, perp_v, _counts = _vq_call(z_flat, embeddings)
    zq3 = _sc_gather(embeddings, idx2)
    z_q = zq3.reshape(z.shape)
    return (z_q, indices, loss_v[0, 0], perp_v[0, 0])
